# baseline (speedup 1.0000x reference)
import functools

import jax
import jax.numpy as jnp
from jax import lax
from jax.experimental import pallas as pl
from jax.experimental.pallas import tpu as pltpu
from jax.experimental.pallas import tpu_sc as plsc

B = 16384
EMB = 32
SHADOW = 32
NC = 2
NS = 16
NW = NC * NS
ROWS_PER_W = B // NW
IDX_CHUNK = 128
NCHUNK = ROWS_PER_W // IDX_CHUNK
NGROUPS = ROWS_PER_W // 16


def _sc_body(u2d, uhi2d, i2d, ihi2d, us_hbm, uemb, ubias16, iemb, ibias16,
             semb, mean_hbm,
             out_hbm,
             uidx_v, uhi_v, iidx_v, ihi_v, us_v, U_v, I_v, S_v, bu_v, bi_v,
             mean_v, out_v, sem):
    wid = lax.axis_index("s") * NC + lax.axis_index("c")
    base = wid * ROWS_PER_W

    pltpu.sync_copy(u2d.at[pl.ds(wid * NCHUNK, NCHUNK)], uidx_v)
    pltpu.sync_copy(uhi2d.at[pl.ds(wid * NCHUNK, NCHUNK)], uhi_v)
    pltpu.sync_copy(i2d.at[pl.ds(wid * NCHUNK, NCHUNK)], iidx_v)
    pltpu.sync_copy(ihi2d.at[pl.ds(wid * NCHUNK, NCHUNK)], ihi_v)
    mean_v[...] = jnp.zeros((16,), jnp.float32)
    pltpu.sync_copy(mean_hbm, mean_v.at[pl.ds(0, 1)])

    copies = [pltpu.async_copy(us_hbm.at[pl.ds(base, ROWS_PER_W)], us_v, sem)]
    for j in range(NCHUNK):
        dst = pl.ds(j * IDX_CHUNK, IDX_CHUNK)
        copies.append(pltpu.async_copy(uemb.at[uidx_v.at[j]], U_v.at[dst], sem))
        copies.append(pltpu.async_copy(iemb.at[iidx_v.at[j]], I_v.at[dst], sem))
        copies.append(pltpu.async_copy(semb.at[iidx_v.at[j]], S_v.at[dst], sem))
        copies.append(pltpu.async_copy(ubias16.at[uhi_v.at[j]], bu_v.at[dst], sem))
        copies.append(pltpu.async_copy(ibias16.at[ihi_v.at[j]], bi_v.at[dst], sem))
    for c in copies:
        c.wait()

    zeros16 = jnp.zeros((16,), jnp.int32)
    mval = mean_v[...][0] + jnp.zeros((16,), jnp.float32)

    def group(g, carry):
        m = g * 16 + lax.iota(jnp.int32, 16)
        mhi = lax.shift_right_logical(m, 7)
        mlo = jnp.bitwise_and(m, 127)
        acc = mval
        kv = zeros16
        for k in range(EMB):
            u = plsc.load_gather(U_v, [m, kv])
            it = plsc.load_gather(I_v, [m, kv])
            us = plsc.load_gather(us_v, [m, kv])
            sh = plsc.load_gather(S_v, [m, kv])
            acc = acc + u * it + us * sh
            if k + 1 < EMB:
                kv = kv + 1
        uvals = plsc.load_gather(uidx_v, [mhi, mlo])
        ivals = plsc.load_gather(iidx_v, [mhi, mlo])
        bu = plsc.load_gather(bu_v, [m, jnp.bitwise_and(uvals, 15)])
        bi = plsc.load_gather(bi_v, [m, jnp.bitwise_and(ivals, 15)])
        out_v[pl.ds(g * 16, 16)] = acc + bu + bi
        return carry

    lax.fori_loop(0, NGROUPS, group, 0)
    pltpu.sync_copy(out_v, out_hbm.at[pl.ds(base, ROWS_PER_W)])


@jax.jit
def _run(u2d, uhi2d, i2d, ihi2d, UserShadow, user_emb, ubias16, item_emb,
         ibias16, shadow_i_emb, mean):
    mesh = plsc.VectorSubcoreMesh(
        core_axis_name="c", subcore_axis_name="s",
        num_cores=NC, num_subcores=NS)
    f = pl.kernel(
        _sc_body,
        out_type=jax.ShapeDtypeStruct((B,), jnp.float32),
        mesh=mesh,
        scratch_types=[
            pltpu.VMEM((NCHUNK, IDX_CHUNK), jnp.int32),
            pltpu.VMEM((NCHUNK, IDX_CHUNK), jnp.int32),
            pltpu.VMEM((NCHUNK, IDX_CHUNK), jnp.int32),
            pltpu.VMEM((NCHUNK, IDX_CHUNK), jnp.int32),
            pltpu.VMEM((ROWS_PER_W, SHADOW), jnp.float32),
            pltpu.VMEM((ROWS_PER_W, EMB), jnp.float32),
            pltpu.VMEM((ROWS_PER_W, EMB), jnp.float32),
            pltpu.VMEM((ROWS_PER_W, SHADOW), jnp.float32),
            pltpu.VMEM((ROWS_PER_W, 16), jnp.float32),
            pltpu.VMEM((ROWS_PER_W, 16), jnp.float32),
            pltpu.VMEM((16,), jnp.float32),
            pltpu.VMEM((ROWS_PER_W,), jnp.float32),
            pltpu.SemaphoreType.DMA,
        ],
        compiler_params=pltpu.CompilerParams(
            needs_layout_passes=False, use_tc_tiling_on_sc=False),
    )
    return f(u2d, uhi2d, i2d, ihi2d, UserShadow, user_emb, ubias16, item_emb,
             ibias16, shadow_i_emb, mean)


def kernel(u_id, i_id, UserShadow, user_emb, user_bias, item_emb, item_bias,
           shadow_i_emb, mean):
    u32 = u_id.astype(jnp.int32)
    i32_ = i_id.astype(jnp.int32)
    u2d = u32.reshape(B // IDX_CHUNK, IDX_CHUNK)
    i2d = i32_.reshape(B // IDX_CHUNK, IDX_CHUNK)
    uhi2d = lax.shift_right_logical(u32, 4).reshape(B // IDX_CHUNK, IDX_CHUNK)
    ihi2d = lax.shift_right_logical(i32_, 4).reshape(B // IDX_CHUNK, IDX_CHUNK)
    ubias16 = user_bias.reshape(-1, 16)
    ibias16 = item_bias.reshape(-1, 16)
    return _run(u2d, uhi2d, i2d, ihi2d, UserShadow, user_emb, ubias16,
                item_emb, ibias16, shadow_i_emb, mean)


# speedup vs baseline: 1.7951x; 1.7951x over previous
import jax
import jax.numpy as jnp
from jax import lax
from jax.experimental import pallas as pl
from jax.experimental.pallas import tpu as pltpu
from jax.experimental.pallas import tpu_sc as plsc

B = 16384
EMB = 32
SHADOW = 32
NC = 2
NS = 16
NW = NC * NS
RPW = B // NW
NGROUPS = RPW // 16
NBROW = 7813


def _sc_body(uid_hbm, iid_hbm, uT, iT, sT, usT, ub128, ib128, mean_hbm,
             out_hbm,
             uids_v, iids_v, uhi_v, ihi_v, us_all,
             uw0, uw1, iw0, iw1, sw0, sw1, bbuf_u, bbuf_i,
             mean_v, out_v, sem, bsem):
    wid = lax.axis_index("s") * NC + lax.axis_index("c")
    base = wid * RPW

    pltpu.sync_copy(uid_hbm.at[pl.ds(base, RPW)], uids_v)
    pltpu.sync_copy(iid_hbm.at[pl.ds(base, RPW)], iids_v)
    mean_v[...] = jnp.zeros((16,), jnp.float32)
    pltpu.sync_copy(mean_hbm, mean_v.at[pl.ds(0, 1)])
    for j in range(4):
        pltpu.sync_copy(usT.at[:, pl.ds(base + j * 128, 128)],
                        us_all.at[pl.ds(j * EMB, EMB), :])

    iota16 = lax.iota(jnp.int32, 16)
    c0 = iota16
    c1 = iota16 + 16
    perms = [jnp.bitwise_xor(iota16, k) for k in (8, 4, 2, 1)]
    zeros16i = jnp.zeros((16,), jnp.int32)
    uwin = (uw0, uw1)
    iwin = (iw0, iw1)
    swin = (sw0, sw1)

    def fire(tabs_u, tabs_i, u, i, p):
        offu = pl.multiple_of(lax.shift_right_logical(u, 7) * 128, 128)
        offi = pl.multiple_of(lax.shift_right_logical(i, 7) * 128, 128)
        cs = [pltpu.async_copy(uT.at[:, pl.ds(offu, 128)], uwin[p], sem),
              pltpu.async_copy(iT.at[:, pl.ds(offi, 128)], iwin[p], sem),
              pltpu.async_copy(sT.at[:, pl.ds(offi, 128)], swin[p], sem)]
        return cs

    def group(g, carry):
        uvec = uids_v[pl.ds(g * 16, 16)]
        ivec = iids_v[pl.ds(g * 16, 16)]
        uhi_v[pl.ds(g * 16, 16)] = lax.shift_right_logical(uvec, 7)
        ihi_v[pl.ds(g * 16, 16)] = lax.shift_right_logical(ivec, 7)

        outv = mean_v[...][0] + jnp.zeros((16,), jnp.float32)
        fire(None, None, uvec[0], ivec[0], 0)
        for l in range(16):
            p = l % 2
            pltpu.make_async_copy(uT.at[:, pl.ds(0, 128)], uwin[p], sem).wait()
            pltpu.make_async_copy(iT.at[:, pl.ds(0, 128)], iwin[p], sem).wait()
            pltpu.make_async_copy(sT.at[:, pl.ds(0, 128)], swin[p], sem).wait()
            if l + 1 < 16:
                fire(None, None, uvec[l + 1], ivec[l + 1], 1 - p)
            u = uvec[l]
            i = ivec[l]
            lane_u = jnp.bitwise_and(u, 127) + zeros16i
            lane_i = jnp.bitwise_and(i, 127) + zeros16i
            b = g * 16 + l
            bhi = lax.shift_right_logical(b, 7) * EMB
            lane_b = jnp.bitwise_and(b, 127) + zeros16i
            u0 = plsc.load_gather(uwin[p], [c0, lane_u])
            u1 = plsc.load_gather(uwin[p], [c1, lane_u])
            i0 = plsc.load_gather(iwin[p], [c0, lane_i])
            i1 = plsc.load_gather(iwin[p], [c1, lane_i])
            s0 = plsc.load_gather(swin[p], [c0, lane_i])
            s1 = plsc.load_gather(swin[p], [c1, lane_i])
            us0 = plsc.load_gather(us_all, [bhi + c0, lane_b])
            us1 = plsc.load_gather(us_all, [bhi + c1, lane_b])
            prod = u0 * i0 + u1 * i1 + us0 * s0 + us1 * s1
            for pm in perms:
                prod = prod + prod.at[pm].get(mode="promise_in_bounds",
                                              unique_indices=True)
            outv = jnp.where(iota16 == l, prod, outv)
        out_v[pl.ds(g * 16, 16)] = outv
        return carry

    lax.fori_loop(0, NGROUPS, group, 0)

    for j in range(4):
        cu = pltpu.async_copy(ub128.at[uhi_v.at[pl.ds(j * 128, 128)]],
                              bbuf_u, bsem)
        ci = pltpu.async_copy(ib128.at[ihi_v.at[pl.ds(j * 128, 128)]],
                              bbuf_i, bsem)
        cu.wait()
        ci.wait()

        def badd(g2, carry):
            m = g2 * 16 + iota16
            s = pl.ds(j * 128 + g2 * 16, 16)
            uvec = uids_v[s]
            ivec = iids_v[s]
            bu = plsc.load_gather(bbuf_u, [m, jnp.bitwise_and(uvec, 127)])
            bi = plsc.load_gather(bbuf_i, [m, jnp.bitwise_and(ivec, 127)])
            out_v[s] = out_v[s] + bu + bi
            return carry

        lax.fori_loop(0, 8, badd, 0)

    pltpu.sync_copy(out_v, out_hbm.at[pl.ds(base, RPW)])


@jax.jit
def _run(uid, iid, uT, iT, sT, usT, ub128, ib128, mean):
    mesh = plsc.VectorSubcoreMesh(
        core_axis_name="c", subcore_axis_name="s",
        num_cores=NC, num_subcores=NS)
    f = pl.kernel(
        _sc_body,
        out_type=jax.ShapeDtypeStruct((B,), jnp.float32),
        mesh=mesh,
        scratch_types=[
            pltpu.VMEM((RPW,), jnp.int32),
            pltpu.VMEM((RPW,), jnp.int32),
            pltpu.VMEM((RPW,), jnp.int32),
            pltpu.VMEM((RPW,), jnp.int32),
            pltpu.VMEM((4 * EMB, 128), jnp.float32),
            pltpu.VMEM((EMB, 128), jnp.float32),
            pltpu.VMEM((EMB, 128), jnp.float32),
            pltpu.VMEM((EMB, 128), jnp.float32),
            pltpu.VMEM((EMB, 128), jnp.float32),
            pltpu.VMEM((EMB, 128), jnp.float32),
            pltpu.VMEM((EMB, 128), jnp.float32),
            pltpu.VMEM((128, 128), jnp.float32),
            pltpu.VMEM((128, 128), jnp.float32),
            pltpu.VMEM((16,), jnp.float32),
            pltpu.VMEM((RPW,), jnp.float32),
            pltpu.SemaphoreType.DMA,
            pltpu.SemaphoreType.DMA,
        ],
        compiler_params=pltpu.CompilerParams(
            needs_layout_passes=False, use_tc_tiling_on_sc=True),
    )
    return f(uid, iid, uT, iT, sT, usT, ub128, ib128, mean)


def kernel(u_id, i_id, UserShadow, user_emb, user_bias, item_emb, item_bias,
           shadow_i_emb, mean):
    uid = u_id.astype(jnp.int32)
    iid = i_id.astype(jnp.int32)
    ub128 = jnp.pad(user_bias.reshape(-1), (0, NBROW * 128 - 1000000)
                    ).reshape(NBROW, 128)
    ib128 = jnp.pad(item_bias.reshape(-1), (0, NBROW * 128 - 1000000)
                    ).reshape(NBROW, 128)
    return _run(uid, iid, user_emb.T, item_emb.T, shadow_i_emb.T,
                UserShadow.T, ub128, ib128, mean)


# speedup vs baseline: 3.2829x; 1.8288x over previous
import jax
import jax.numpy as jnp
from jax import lax
from jax.experimental import pallas as pl
from jax.experimental.pallas import tpu as pltpu
from jax.experimental.pallas import tpu_sc as plsc

B = 16384
EMB = 32
SHADOW = 32
NC = 2
NS = 16
NW = NC * NS
RPW = B // NW
NGROUPS = RPW // 16
NBROW = 7813
DEPTH = 4
IDPAD = RPW + 32


def _sc_body(uid_hbm, iid_hbm, uT, iT, sT, usT, ub128, ib128, mean_hbm,
             out_hbm,
             uids_v, iids_v, uhi_v, ihi_v, us_all,
             uw, iw, sw, bbuf_u, bbuf_i,
             mean_v, out_v, sem, bsem):
    wid = lax.axis_index("s") * NC + lax.axis_index("c")
    base = wid * RPW

    pltpu.sync_copy(uid_hbm.at[pl.ds(base, RPW)], uids_v.at[pl.ds(0, RPW)])
    pltpu.sync_copy(iid_hbm.at[pl.ds(base, RPW)], iids_v.at[pl.ds(0, RPW)])
    zeros16i = jnp.zeros((16,), jnp.int32)
    uids_v[pl.ds(RPW, 16)] = zeros16i
    uids_v[pl.ds(RPW + 16, 16)] = zeros16i
    iids_v[pl.ds(RPW, 16)] = zeros16i
    iids_v[pl.ds(RPW + 16, 16)] = zeros16i
    mean_v[...] = jnp.zeros((16,), jnp.float32)
    pltpu.sync_copy(mean_hbm, mean_v.at[pl.ds(0, 1)])
    for j in range(4):
        pltpu.sync_copy(usT.at[:, pl.ds(base + j * 128, 128)],
                        us_all.at[pl.ds(j * EMB, EMB), :])

    iota16 = lax.iota(jnp.int32, 16)
    c0 = iota16
    c1 = iota16 + 16
    perms = [jnp.bitwise_xor(iota16, k) for k in (8, 4, 2, 1)]

    def fire(u, i, p):
        offu = pl.multiple_of(lax.shift_right_logical(u, 7) * 128, 128)
        offi = pl.multiple_of(lax.shift_right_logical(i, 7) * 128, 128)
        pltpu.async_copy(uT.at[:, pl.ds(offu, 128)], uw.at[p], sem)
        pltpu.async_copy(iT.at[:, pl.ds(offi, 128)], iw.at[p], sem)
        pltpu.async_copy(sT.at[:, pl.ds(offi, 128)], sw.at[p], sem)

    def wait(p):
        pltpu.make_async_copy(uT.at[:, pl.ds(0, 128)], uw.at[p], sem).wait()
        pltpu.make_async_copy(iT.at[:, pl.ds(0, 128)], iw.at[p], sem).wait()
        pltpu.make_async_copy(sT.at[:, pl.ds(0, 128)], sw.at[p], sem).wait()

    uvec0 = uids_v[pl.ds(0, 16)]
    ivec0 = iids_v[pl.ds(0, 16)]
    for l in range(DEPTH - 1):
        fire(uvec0[l], ivec0[l], l)

    mval = mean_v[...][0]

    def group(g, carry):
        uvec = uids_v[pl.ds(g * 16, 16)]
        ivec = iids_v[pl.ds(g * 16, 16)]
        uvecn = uids_v[pl.ds(g * 16 + 16, 16)]
        ivecn = iids_v[pl.ds(g * 16 + 16, 16)]
        uhi_v[pl.ds(g * 16, 16)] = lax.shift_right_logical(uvec, 7)
        ihi_v[pl.ds(g * 16, 16)] = lax.shift_right_logical(ivec, 7)

        outv = mval + jnp.zeros((16,), jnp.float32)
        for l in range(16):
            p = l % DEPTH
            wait(p)
            la = l + DEPTH - 1
            if la < 16:
                fire(uvec[la], ivec[la], la % DEPTH)
            else:
                fire(uvecn[la - 16], ivecn[la - 16], la % DEPTH)
            u = uvec[l]
            i = ivec[l]
            lane_u = jnp.bitwise_and(u, 127) + zeros16i
            lane_i = jnp.bitwise_and(i, 127) + zeros16i
            b = g * 16 + l
            bhi = lax.shift_right_logical(b, 7) * EMB
            lane_b = jnp.bitwise_and(b, 127) + zeros16i
            u0 = plsc.load_gather(uw.at[p], [c0, lane_u])
            u1 = plsc.load_gather(uw.at[p], [c1, lane_u])
            i0 = plsc.load_gather(iw.at[p], [c0, lane_i])
            i1 = plsc.load_gather(iw.at[p], [c1, lane_i])
            s0 = plsc.load_gather(sw.at[p], [c0, lane_i])
            s1 = plsc.load_gather(sw.at[p], [c1, lane_i])
            us0 = plsc.load_gather(us_all, [bhi + c0, lane_b])
            us1 = plsc.load_gather(us_all, [bhi + c1, lane_b])
            prod = u0 * i0 + u1 * i1 + us0 * s0 + us1 * s1
            for pm in perms:
                prod = prod + prod.at[pm].get(mode="promise_in_bounds",
                                              unique_indices=True)
            outv = jnp.where(iota16 == l, prod, outv)
        out_v[pl.ds(g * 16, 16)] = outv
        return carry

    lax.fori_loop(0, NGROUPS, group, 0)
    for l in range(DEPTH - 1):
        wait((RPW + l) % DEPTH)

    for j in range(4):
        cu = pltpu.async_copy(ub128.at[uhi_v.at[pl.ds(j * 128, 128)]],
                              bbuf_u, bsem)
        ci = pltpu.async_copy(ib128.at[ihi_v.at[pl.ds(j * 128, 128)]],
                              bbuf_i, bsem)
        cu.wait()
        ci.wait()

        def badd(g2, carry):
            m = g2 * 16 + iota16
            s = pl.ds(j * 128 + g2 * 16, 16)
            uvec = uids_v[s]
            ivec = iids_v[s]
            bu = plsc.load_gather(bbuf_u, [m, jnp.bitwise_and(uvec, 127)])
            bi = plsc.load_gather(bbuf_i, [m, jnp.bitwise_and(ivec, 127)])
            out_v[s] = out_v[s] + bu + bi
            return carry

        lax.fori_loop(0, 8, badd, 0)

    pltpu.sync_copy(out_v, out_hbm.at[pl.ds(base, RPW)])


@jax.jit
def _run(uid, iid, uT, iT, sT, usT, ub128, ib128, mean):
    mesh = plsc.VectorSubcoreMesh(
        core_axis_name="c", subcore_axis_name="s",
        num_cores=NC, num_subcores=NS)
    f = pl.kernel(
        _sc_body,
        out_type=jax.ShapeDtypeStruct((B,), jnp.float32),
        mesh=mesh,
        scratch_types=[
            pltpu.VMEM((IDPAD,), jnp.int32),
            pltpu.VMEM((IDPAD,), jnp.int32),
            pltpu.VMEM((RPW,), jnp.int32),
            pltpu.VMEM((RPW,), jnp.int32),
            pltpu.VMEM((4 * EMB, 128), jnp.float32),
            pltpu.VMEM((DEPTH, EMB, 128), jnp.float32),
            pltpu.VMEM((DEPTH, EMB, 128), jnp.float32),
            pltpu.VMEM((DEPTH, EMB, 128), jnp.float32),
            pltpu.VMEM((128, 128), jnp.float32),
            pltpu.VMEM((128, 128), jnp.float32),
            pltpu.VMEM((16,), jnp.float32),
            pltpu.VMEM((RPW,), jnp.float32),
            pltpu.SemaphoreType.DMA,
            pltpu.SemaphoreType.DMA,
        ],
        compiler_params=pltpu.CompilerParams(
            needs_layout_passes=False, use_tc_tiling_on_sc=True),
    )
    return f(uid, iid, uT, iT, sT, usT, ub128, ib128, mean)


def kernel(u_id, i_id, UserShadow, user_emb, user_bias, item_emb, item_bias,
           shadow_i_emb, mean):
    uid = u_id.astype(jnp.int32)
    iid = i_id.astype(jnp.int32)
    ub128 = jnp.pad(user_bias.reshape(-1), (0, NBROW * 128 - 1000000)
                    ).reshape(NBROW, 128)
    ib128 = jnp.pad(item_bias.reshape(-1), (0, NBROW * 128 - 1000000)
                    ).reshape(NBROW, 128)
    return _run(uid, iid, user_emb.T, item_emb.T, shadow_i_emb.T,
                UserShadow.T, ub128, ib128, mean)
